# Initial kernel scaffold; baseline (speedup 1.0000x reference)
import jax
import jax.numpy as jnp
from jax.experimental import pallas as pl


def kernel(x, adj_t, W_rel1, b_rel1, W_root1, W_rel2, b_rel2, W_root2):
    raise NotImplementedError("write your pallas kernel here")


# speedup vs baseline: 5.7651x; 5.7651x over previous
import jax
import jax.numpy as jnp
from jax import lax
from jax.experimental import pallas as pl
from jax.experimental.pallas import tpu as pltpu
from jax.experimental.pallas import tpu_sc as plsc

N = 10000
D = 128
L1 = 64
E = 320000

NC, NS = 2, 16
NW = NC * NS
CHUNK = 128
K = 79
EPW = K * CHUNK
E_PAD = NW * EPW
N_PAD = 10112
RPT = N_PAD // NS

_f32 = jnp.float32


def _conv1_body(p_ref, x_ref, wr1_ref, b1_ref, wo1_ref, x1_ref):
    aggr = p_ref[0, :N, :] + p_ref[1, :N, :]
    z = (jnp.dot(aggr, wr1_ref[...], preferred_element_type=_f32)
         + b1_ref[...]
         + jnp.dot(x_ref[...], wo1_ref[...], preferred_element_type=_f32))
    x1_ref[...] = jnp.where(z >= 0, z, 0.2 * z)


def _final_body(p2_ref, x1_ref, wr2_ref, b2_ref, wo2_ref, x_ref, o_ref):
    aggr2 = p2_ref[0, :N, :] + p2_ref[1, :N, :]
    gate = (jnp.dot(aggr2, wr2_ref[...], preferred_element_type=_f32)
            + b2_ref[0, 0]
            + jnp.dot(x1_ref[...], wo2_ref[...], preferred_element_type=_f32))
    m = jnp.max(gate)
    e = jnp.exp(gate - m)
    w = e / jnp.sum(e)
    o_ref[...] = jnp.sum(w * x_ref[...], axis=0, keepdims=True)


def _make_seg_body(width):
    n_full, rem = divmod(RPT, CHUNK)

    def body(t_h, src_h, dst_h, out_h, acc_s, srcb, dstb, rows, sem):
        c = lax.axis_index("c")
        s = lax.axis_index("s")
        wid = c * NS + s

        def _z(i, _):
            rows[i // (width // 16), pl.ds((i % (width // 16)) * 16, 16)] = (
                jnp.zeros((16,), _f32))
            return 0
        lax.fori_loop(0, CHUNK * (width // 16), _z, 0)
        for kk in range(n_full):
            pltpu.sync_copy(rows, acc_s.at[pl.ds(s * RPT + kk * CHUNK, CHUNK)])
        if rem:
            pltpu.sync_copy(rows.at[pl.ds(0, rem)],
                            acc_s.at[pl.ds(s * RPT + n_full * CHUNK, rem)])
        plsc.subcore_barrier()

        pltpu.sync_copy(src_h.at[wid], srcb)
        pltpu.sync_copy(dst_h.at[wid], dstb)

        def _step(j, _):
            pltpu.async_copy(t_h.at[srcb.at[j]], rows, sem).wait()
            pltpu.sync_copy(rows, acc_s.at[dstb.at[j]], add=True)
            return 0
        lax.fori_loop(0, K, _step, 0)
        plsc.subcore_barrier()

        for kk in range(n_full):
            pltpu.sync_copy(acc_s.at[pl.ds(s * RPT + kk * CHUNK, CHUNK)], rows)
            pltpu.sync_copy(rows, out_h.at[c, pl.ds(s * RPT + kk * CHUNK, CHUNK)])
        if rem:
            pltpu.sync_copy(acc_s.at[pl.ds(s * RPT + n_full * CHUNK, rem)],
                            rows.at[pl.ds(0, rem)])
            pltpu.sync_copy(rows.at[pl.ds(0, rem)],
                            out_h.at[c, pl.ds(s * RPT + n_full * CHUNK, rem)])

    return body


def _make_seg(width):
    return pl.kernel(
        _make_seg_body(width),
        out_type=jax.ShapeDtypeStruct((NC, N_PAD, width), _f32),
        mesh=plsc.VectorSubcoreMesh(core_axis_name="c", subcore_axis_name="s"),
        scratch_types=[
            pltpu.VMEM_SHARED((N_PAD, width), _f32),
            pltpu.VMEM((K, CHUNK), jnp.int32),
            pltpu.VMEM((K, CHUNK), jnp.int32),
            pltpu.VMEM((CHUNK, width), _f32),
            pltpu.SemaphoreType.DMA,
        ],
        compiler_params=pltpu.CompilerParams(use_tc_tiling_on_sc=False),
    )


_seg128 = _make_seg(D)
_seg64 = _make_seg(L1)

_conv1 = pl.pallas_call(
    _conv1_body,
    out_shape=jax.ShapeDtypeStruct((N, L1), _f32),
)

_final = pl.pallas_call(
    _final_body,
    out_shape=jax.ShapeDtypeStruct((1, D), _f32),
)


def kernel(x, adj_t, W_rel1, b_rel1, W_root1, W_rel2, b_rel2, W_root2):
    src = adj_t[0].astype(jnp.int32)
    dst = adj_t[1].astype(jnp.int32)
    pad = E_PAD - E
    src_p = jnp.concatenate([src, jnp.zeros((pad,), jnp.int32)]).reshape(NW, K, CHUNK)
    dst_p = jnp.concatenate([dst, jnp.full((pad,), N, jnp.int32)]).reshape(NW, K, CHUNK)

    parts = _seg128(x, src_p, dst_p)
    x1 = _conv1(parts, x, W_rel1, b_rel1.reshape(1, L1), W_root1)
    parts2 = _seg64(x1, src_p, dst_p)
    out = _final(parts2, x1, W_rel2, b_rel2.reshape(1, 1), W_root2, x)
    return out
